# baseline (speedup 1.0000x reference)
import functools

import jax
import jax.numpy as jnp
from jax.experimental import pallas as pl
from jax.experimental.pallas import tpu as pltpu

N = 10000
E = 320000
D = 128
R = 16
K = 11
G = 64

CH = 2000
NCH = E // CH


def _embed_body(x_ref, w_ref, b_ref, out_ref):
    out_ref[...] = jnp.dot(x_ref[...], w_ref[...],
                           preferred_element_type=jnp.float32) + b_ref[...][None, :]


def _cnt_body(dst_ref, typ_ref, cnt_ref):
    pc = pl.program_id(0)

    @pl.when(pc == 0)
    def _():
        cnt_ref[...] = jnp.zeros_like(cnt_ref)

    iota = jax.lax.broadcasted_iota(jnp.int32, (1, R), 1)

    def body(i, _):
        d = dst_ref[0, 0, i]
        t = typ_ref[0, 0, i]
        oh = (iota == t).astype(jnp.float32)
        cnt_ref[pl.ds(d, 1), :] += oh
        return 0

    jax.lax.fori_loop(0, CH, body, 0)


def _rgcn_body(h_ref, src_ref, dst_ref, typ_ref, cnt_ref, wrel_ref,
               wroot_ref, b_ref, out_ref, acc_ref, *, relu_out):
    pr = pl.program_id(0)
    pc = pl.program_id(1)

    @pl.when(jnp.logical_and(pr == 0, pc == 0))
    def _():
        out_ref[...] = jnp.dot(h_ref[...], wroot_ref[...],
                               preferred_element_type=jnp.float32) + b_ref[...][None, :]

    @pl.when(pc == 0)
    def _():
        acc_ref[...] = jnp.zeros_like(acc_ref)

    def body(i, _):
        t = typ_ref[0, 0, i]

        @pl.when(t == pr)
        def _():
            s = src_ref[0, 0, i]
            d = dst_ref[0, 0, i]
            acc_ref[pl.ds(d, 1), :] += h_ref[pl.ds(s, 1), :]

        return 0

    jax.lax.fori_loop(0, CH, body, 0)

    @pl.when(pc == NCH - 1)
    def _():
        roh = jax.lax.broadcasted_iota(jnp.int32, (R, 1), 0)
        sel = (roh == pr).astype(jnp.float32)
        col = jnp.dot(cnt_ref[...], sel, preferred_element_type=jnp.float32)
        nrm = 1.0 / jnp.maximum(col, 1.0)
        out_ref[...] += jnp.dot(acc_ref[...] * nrm, wrel_ref[0],
                                preferred_element_type=jnp.float32)

    if relu_out:
        @pl.when(jnp.logical_and(pr == R - 1, pc == NCH - 1))
        def _():
            out_ref[...] = jnp.maximum(out_ref[...], 0.0)


def _mf_body(h_ref, src_ref, dst_ref, cnt_ref, wl_ref, bl_ref, wr_ref,
             out_ref, agg_ref, *, relu_out):
    pc = pl.program_id(0)

    @pl.when(pc == 0)
    def _():
        agg_ref[...] = jnp.zeros_like(agg_ref)

    def body(i, _):
        s = src_ref[0, 0, i]
        d = dst_ref[0, 0, i]
        agg_ref[pl.ds(d, 1), :] += h_ref[pl.ds(s, 1), :]
        return 0

    jax.lax.fori_loop(0, CH, body, 0)

    @pl.when(pc == NCH - 1)
    def _():
        ones = jnp.ones((R, 1), dtype=jnp.float32)
        deg = jnp.dot(cnt_ref[...], ones, preferred_element_type=jnp.float32)
        deg = jnp.minimum(deg, float(K - 1))
        h = h_ref[...]
        agg = agg_ref[...]
        acc = jnp.zeros((N, D), dtype=jnp.float32)
        for k in range(K):
            mk = (deg == float(k)).astype(jnp.float32)
            hk = (jnp.dot(h, wl_ref[k], preferred_element_type=jnp.float32)
                  + bl_ref[k][None, :]
                  + jnp.dot(agg, wr_ref[k], preferred_element_type=jnp.float32))
            acc = acc + mk * hk
        if relu_out:
            acc = jnp.maximum(acc, 0.0)
        out_ref[...] = acc


def _pool_body(h_ref, b3_ref, w1_ref, b1_ref, w2_ref, b2_ref, out_ref, pool_ref):
    pool_ref[...] = jnp.zeros_like(pool_ref)

    def body(i, _):
        g = b3_ref[0, 0, i]
        pool_ref[pl.ds(g, 1), :] += h_ref[pl.ds(i, 1), :]
        return 0

    jax.lax.fori_loop(0, N, body, 0)

    h2 = jnp.maximum(jnp.dot(pool_ref[...], w1_ref[...],
                             preferred_element_type=jnp.float32)
                     + b1_ref[...][None, :], 0.0)
    out_ref[...] = jnp.dot(h2, w2_ref[...],
                           preferred_element_type=jnp.float32) + b2_ref[...][None, :]


def _full(shape):
    nd = len(shape)
    return pl.BlockSpec(shape, lambda *_: (0,) * nd)


def _edge_spec(nargs):
    if nargs == 2:
        return pl.BlockSpec((1, 1, CH), lambda r, c: (c, 0, 0),
                            memory_space=pltpu.SMEM)
    return pl.BlockSpec((1, 1, CH), lambda c: (c, 0, 0),
                        memory_space=pltpu.SMEM)


def _rgcn_call(h, src3, dst3, typ3, cnt, wrel, wroot, b, relu_out):
    return pl.pallas_call(
        functools.partial(_rgcn_body, relu_out=relu_out),
        grid=(R, NCH),
        in_specs=[
            _full((N, D)),
            _edge_spec(2), _edge_spec(2), _edge_spec(2),
            _full((N, R)),
            pl.BlockSpec((1, D, D), lambda r, c: (r, 0, 0)),
            _full((D, D)),
            _full((D,)),
        ],
        out_specs=_full((N, D)),
        out_shape=jax.ShapeDtypeStruct((N, D), jnp.float32),
        scratch_shapes=[pltpu.VMEM((N, D), jnp.float32)],
    )(h, src3, dst3, typ3, cnt, wrel, wroot, b)


def _mf_call(h, src3, dst3, cnt, wl, bl, wr, relu_out):
    return pl.pallas_call(
        functools.partial(_mf_body, relu_out=relu_out),
        grid=(NCH,),
        in_specs=[
            _full((N, D)),
            _edge_spec(1), _edge_spec(1),
            _full((N, R)),
            _full((K, D, D)),
            _full((K, D)),
            _full((K, D, D)),
        ],
        out_specs=_full((N, D)),
        out_shape=jax.ShapeDtypeStruct((N, D), jnp.float32),
        scratch_shapes=[pltpu.VMEM((N, D), jnp.float32)],
    )(h, src3, dst3, cnt, wl, bl, wr)


def kernel(x, edge_index, edge_attr, batch_idx, w_embed, b_embed,
           rgcn_rel_0, rgcn_root_0, rgcn_b_0, mf_wl_0, mf_bl_0, mf_wr_0,
           rgcn_rel_1, rgcn_root_1, rgcn_b_1, mf_wl_1, mf_bl_1, mf_wr_1,
           w1, b1, w2, b2):
    src3 = edge_index[0].reshape(NCH, 1, CH)
    dst3 = edge_index[1].reshape(NCH, 1, CH)
    typ3 = edge_attr.reshape(NCH, 1, CH)
    batch3 = batch_idx.reshape(1, 1, N)

    h0 = pl.pallas_call(
        _embed_body,
        grid=(1,),
        in_specs=[_full((N, D)), _full((D, D)), _full((D,))],
        out_specs=_full((N, D)),
        out_shape=jax.ShapeDtypeStruct((N, D), jnp.float32),
    )(x, w_embed, b_embed)

    cnt = pl.pallas_call(
        _cnt_body,
        grid=(NCH,),
        in_specs=[_edge_spec(1), _edge_spec(1)],
        out_specs=_full((N, R)),
        out_shape=jax.ShapeDtypeStruct((N, R), jnp.float32),
    )(dst3, typ3)

    h = _rgcn_call(h0, src3, dst3, typ3, cnt, rgcn_rel_0, rgcn_root_0,
                   rgcn_b_0, relu_out=True)
    h = _mf_call(h, src3, dst3, cnt, mf_wl_0, mf_bl_0, mf_wr_0, relu_out=True)
    h = _rgcn_call(h, src3, dst3, typ3, cnt, rgcn_rel_1, rgcn_root_1,
                   rgcn_b_1, relu_out=True)
    h = _mf_call(h, src3, dst3, cnt, mf_wl_1, mf_bl_1, mf_wr_1, relu_out=False)

    out = pl.pallas_call(
        _pool_body,
        grid=(1,),
        in_specs=[
            _full((N, D)),
            pl.BlockSpec((1, 1, N), lambda *_: (0, 0, 0), memory_space=pltpu.SMEM),
            _full((D, D)), _full((D,)), _full((D, 1)), _full((1,)),
        ],
        out_specs=_full((G, 1)),
        out_shape=jax.ShapeDtypeStruct((G, 1), jnp.float32),
        scratch_shapes=[pltpu.VMEM((G, D), jnp.float32)],
    )(h, batch3, w1, b1, w2, b2)

    return out


# speedup vs baseline: 3.3690x; 3.3690x over previous
import functools

import jax
import jax.numpy as jnp
from jax.experimental import pallas as pl
from jax.experimental.pallas import tpu as pltpu

N = 10000
E = 320000
D = 128
R = 16
K = 11
G = 64

CH = 2000
NCH = E // CH


def _embed_body(x_ref, w_ref, b_ref, out_ref):
    out_ref[...] = jnp.dot(x_ref[...], w_ref[...],
                           preferred_element_type=jnp.float32) + b_ref[...][None, :]


def _cnt_body(dst_ref, typ_ref, cnt_ref):
    pc = pl.program_id(0)

    @pl.when(pc == 0)
    def _():
        cnt_ref[...] = jnp.zeros_like(cnt_ref)

    iota = jax.lax.broadcasted_iota(jnp.int32, (1, R), 1)

    def body(i, _):
        d = dst_ref[0, 0, i]
        t = typ_ref[0, 0, i]
        oh = (iota == t).astype(jnp.float32)
        cnt_ref[pl.ds(d, 1), :] += oh
        return 0

    jax.lax.fori_loop(0, CH, body, 0)


GR = 4
NG = R // GR


def _rgcn_body(h_ref, src_ref, dst_ref, typ_ref, cnt_ref, wrel_ref,
               wroot_ref, b_ref, out_ref, acc_ref, *, relu_out):
    pg = pl.program_id(0)
    pc = pl.program_id(1)

    @pl.when(jnp.logical_and(pg == 0, pc == 0))
    def _():
        out_ref[...] = jnp.dot(h_ref[...], wroot_ref[...],
                               preferred_element_type=jnp.float32) + b_ref[...][None, :]

    @pl.when(pc == 0)
    def _():
        acc_ref[...] = jnp.zeros_like(acc_ref)

    base = pg * GR

    def body(i, _):
        t = typ_ref[0, 0, i] - base

        @pl.when(jnp.logical_and(t >= 0, t < GR))
        def _():
            s = src_ref[0, 0, i]
            d = dst_ref[0, 0, i]
            acc_ref[pl.ds(t * N + d, 1), :] += h_ref[pl.ds(s, 1), :]

        return 0

    jax.lax.fori_loop(0, CH, body, 0)

    @pl.when(pc == NCH - 1)
    def _():
        riota = jax.lax.broadcasted_iota(jnp.int32, (R, 1), 0)
        for r in range(GR):
            sel = (riota == base + r).astype(jnp.float32)
            col = jnp.dot(cnt_ref[...], sel, preferred_element_type=jnp.float32)
            nrm = 1.0 / jnp.maximum(col, 1.0)
            out_ref[...] += jnp.dot(acc_ref[pl.ds(r * N, N), :] * nrm,
                                    wrel_ref[r],
                                    preferred_element_type=jnp.float32)

    if relu_out:
        @pl.when(jnp.logical_and(pg == NG - 1, pc == NCH - 1))
        def _():
            out_ref[...] = jnp.maximum(out_ref[...], 0.0)


def _mf_body(h_ref, src_ref, dst_ref, cnt_ref, wl_ref, bl_ref, wr_ref,
             out_ref, agg_ref, *, relu_out):
    pc = pl.program_id(0)

    @pl.when(pc == 0)
    def _():
        agg_ref[...] = jnp.zeros_like(agg_ref)

    def body(i, _):
        s = src_ref[0, 0, i]
        d = dst_ref[0, 0, i]
        agg_ref[pl.ds(d, 1), :] += h_ref[pl.ds(s, 1), :]
        return 0

    jax.lax.fori_loop(0, CH, body, 0)

    @pl.when(pc == NCH - 1)
    def _():
        ones = jnp.ones((R, 1), dtype=jnp.float32)
        deg = jnp.dot(cnt_ref[...], ones, preferred_element_type=jnp.float32)
        deg = jnp.minimum(deg, float(K - 1))
        h = h_ref[...]
        agg = agg_ref[...]
        acc = jnp.zeros((N, D), dtype=jnp.float32)
        for k in range(K):
            mk = (deg == float(k)).astype(jnp.float32)
            hk = (jnp.dot(h, wl_ref[k], preferred_element_type=jnp.float32)
                  + bl_ref[k][None, :]
                  + jnp.dot(agg, wr_ref[k], preferred_element_type=jnp.float32))
            acc = acc + mk * hk
        if relu_out:
            acc = jnp.maximum(acc, 0.0)
        out_ref[...] = acc


def _pool_body(h_ref, b3_ref, w1_ref, b1_ref, w2_ref, b2_ref, out_ref, pool_ref):
    pool_ref[...] = jnp.zeros_like(pool_ref)

    def body(i, _):
        g = b3_ref[0, 0, i]
        pool_ref[pl.ds(g, 1), :] += h_ref[pl.ds(i, 1), :]
        return 0

    jax.lax.fori_loop(0, N, body, 0)

    h2 = jnp.maximum(jnp.dot(pool_ref[...], w1_ref[...],
                             preferred_element_type=jnp.float32)
                     + b1_ref[...][None, :], 0.0)
    out_ref[...] = jnp.dot(h2, w2_ref[...],
                           preferred_element_type=jnp.float32) + b2_ref[...][None, :]


def _full(shape):
    nd = len(shape)
    return pl.BlockSpec(shape, lambda *_: (0,) * nd)


def _edge_spec(nargs):
    if nargs == 2:
        return pl.BlockSpec((1, 1, CH), lambda r, c: (c, 0, 0),
                            memory_space=pltpu.SMEM)
    return pl.BlockSpec((1, 1, CH), lambda c: (c, 0, 0),
                        memory_space=pltpu.SMEM)


def _rgcn_call(h, src3, dst3, typ3, cnt, wrel, wroot, b, relu_out):
    return pl.pallas_call(
        functools.partial(_rgcn_body, relu_out=relu_out),
        grid=(NG, NCH),
        in_specs=[
            _full((N, D)),
            _edge_spec(2), _edge_spec(2), _edge_spec(2),
            _full((N, R)),
            pl.BlockSpec((GR, D, D), lambda g, c: (g, 0, 0)),
            _full((D, D)),
            _full((D,)),
        ],
        out_specs=_full((N, D)),
        out_shape=jax.ShapeDtypeStruct((N, D), jnp.float32),
        scratch_shapes=[pltpu.VMEM((GR * N, D), jnp.float32)],
    )(h, src3, dst3, typ3, cnt, wrel, wroot, b)


def _mf_call(h, src3, dst3, cnt, wl, bl, wr, relu_out):
    return pl.pallas_call(
        functools.partial(_mf_body, relu_out=relu_out),
        grid=(NCH,),
        in_specs=[
            _full((N, D)),
            _edge_spec(1), _edge_spec(1),
            _full((N, R)),
            _full((K, D, D)),
            _full((K, D)),
            _full((K, D, D)),
        ],
        out_specs=_full((N, D)),
        out_shape=jax.ShapeDtypeStruct((N, D), jnp.float32),
        scratch_shapes=[pltpu.VMEM((N, D), jnp.float32)],
    )(h, src3, dst3, cnt, wl, bl, wr)


def kernel(x, edge_index, edge_attr, batch_idx, w_embed, b_embed,
           rgcn_rel_0, rgcn_root_0, rgcn_b_0, mf_wl_0, mf_bl_0, mf_wr_0,
           rgcn_rel_1, rgcn_root_1, rgcn_b_1, mf_wl_1, mf_bl_1, mf_wr_1,
           w1, b1, w2, b2):
    src3 = edge_index[0].reshape(NCH, 1, CH)
    dst3 = edge_index[1].reshape(NCH, 1, CH)
    typ3 = edge_attr.reshape(NCH, 1, CH)
    batch3 = batch_idx.reshape(1, 1, N)

    h0 = pl.pallas_call(
        _embed_body,
        grid=(1,),
        in_specs=[_full((N, D)), _full((D, D)), _full((D,))],
        out_specs=_full((N, D)),
        out_shape=jax.ShapeDtypeStruct((N, D), jnp.float32),
    )(x, w_embed, b_embed)

    cnt = pl.pallas_call(
        _cnt_body,
        grid=(NCH,),
        in_specs=[_edge_spec(1), _edge_spec(1)],
        out_specs=_full((N, R)),
        out_shape=jax.ShapeDtypeStruct((N, R), jnp.float32),
    )(dst3, typ3)

    h = _rgcn_call(h0, src3, dst3, typ3, cnt, rgcn_rel_0, rgcn_root_0,
                   rgcn_b_0, relu_out=True)
    h = _mf_call(h, src3, dst3, cnt, mf_wl_0, mf_bl_0, mf_wr_0, relu_out=True)
    h = _rgcn_call(h, src3, dst3, typ3, cnt, rgcn_rel_1, rgcn_root_1,
                   rgcn_b_1, relu_out=True)
    h = _mf_call(h, src3, dst3, cnt, mf_wl_1, mf_bl_1, mf_wr_1, relu_out=False)

    out = pl.pallas_call(
        _pool_body,
        grid=(1,),
        in_specs=[
            _full((N, D)),
            pl.BlockSpec((1, 1, N), lambda *_: (0, 0, 0), memory_space=pltpu.SMEM),
            _full((D, D)), _full((D,)), _full((D, 1)), _full((1,)),
        ],
        out_specs=_full((G, 1)),
        out_shape=jax.ShapeDtypeStruct((G, 1), jnp.float32),
        scratch_shapes=[pltpu.VMEM((G, D), jnp.float32)],
    )(h, batch3, w1, b1, w2, b2)

    return out
